# baseline (speedup 1.0000x reference)
import functools

import jax
import jax.numpy as jnp
from jax import lax
from jax.experimental import pallas as pl
from jax.experimental.pallas import tpu as pltpu
from jax.experimental.pallas import tpu_sc as plsc

NC = 2
NS = 16
NW = NC * NS
L = 16

BLK = 256
CH = 2048
GB = 112
NREGP = 176

_SC_PARAMS = dict(
    compiler_params=pltpu.CompilerParams(needs_layout_passes=False))


def _take16(v, idx):
    return lax.gather(
        v, idx[:, None],
        lax.GatherDimensionNumbers(offset_dims=(), collapsed_slice_dims=(0,),
                                   start_index_map=(0,)),
        (1,), mode=lax.GatherScatterMode.PROMISE_IN_BOUNDS)


def _run_length_split(s, pos, pos_next, pos_prev):
    is_last = (s != _take16(s, pos_next)) | (pos == L - 1)
    is_first = (s != _take16(s, pos_prev)) | (pos == 0)
    fpos = plsc.cummax(jnp.where(is_first, pos, -1))
    return pos - fpos, is_last


def _sc_count(col, et, T, N, NB):
    E = col.shape[0]
    EPW = E // NW
    DSZ = T * N + L
    mesh = plsc.VectorSubcoreMesh(core_axis_name="c", subcore_axis_name="s",
                                  num_cores=NC, num_subcores=NS)

    @functools.partial(
        pl.kernel,
        out_type=(jax.ShapeDtypeStruct((NW, DSZ), jnp.int32),
                  jax.ShapeDtypeStruct((NW, NREGP), jnp.int32)),
        mesh=mesh,
        scratch_types=[
            pltpu.VMEM((EPW,), jnp.int32),
            pltpu.VMEM((EPW,), jnp.int32),
            pltpu.VMEM((DSZ,), jnp.int32),
            pltpu.VMEM((NREGP,), jnp.int32),
        ],
        **_SC_PARAMS,
    )
    def k(col_hbm, et_hbm, deg_hbm, reg_hbm, colv, etv, degv, regv):
        wid = lax.axis_index("s") * NC + lax.axis_index("c")
        base = wid * EPW
        pltpu.sync_copy(col_hbm.at[pl.ds(base, EPW)], colv)
        pltpu.sync_copy(et_hbm.at[pl.ds(base, EPW)], etv)

        zero = jnp.zeros((L,), jnp.int32)

        def zd(i, _):
            degv[pl.ds(i * L, L)] = zero
            return 0

        lax.fori_loop(0, DSZ // L, zd, 0)
        for i in range(NREGP // L):
            regv[pl.ds(i * L, L)] = zero

        pos = lax.iota(jnp.int32, L)
        pos_next = jnp.minimum(pos + 1, L - 1)
        pos_prev = jnp.maximum(pos - 1, 0)

        def hist(tab, keys):
            s, _ = plsc.sort_key_val(keys, keys)
            rank, is_last = _run_length_split(s, pos, pos_next, pos_prev)
            old = plsc.load_gather(tab, [s])
            plsc.store_scatter(tab, [s], old + rank + 1, mask=is_last)

        def sbody(i, _):
            cv = colv[pl.ds(i * L, L)]
            tv = etv[pl.ds(i * L, L)]
            hist(degv, tv * N + cv)
            gg = jnp.where(tv < T, tv * NB + lax.shift_right_logical(cv, 8),
                           T * NB)
            hist(regv, gg)
            return 0

        lax.fori_loop(0, EPW // L, sbody, 0)
        pltpu.sync_copy(degv, deg_hbm.at[wid])
        pltpu.sync_copy(regv, reg_hbm.at[wid])

    return k(col, et)


def _sc_bucket(row, col, et, regp, T, NB, NPAD, EXT):
    E = row.shape[0]
    EPW = E // NW
    NGRP = EPW // L
    NROW = (NGRP * L + 127) // 128
    mesh = plsc.VectorSubcoreMesh(core_axis_name="c", subcore_axis_name="s",
                                  num_cores=NC, num_subcores=NS)

    @functools.partial(
        pl.kernel,
        out_type=(jax.ShapeDtypeStruct((EXT,), jnp.int32),
                  jax.ShapeDtypeStruct((256,), jnp.int32)),
        mesh=mesh,
        scratch_types=[
            pltpu.VMEM((EPW,), jnp.int32),
            pltpu.VMEM((EPW,), jnp.int32),
            pltpu.VMEM((EPW,), jnp.int32),
            pltpu.VMEM((NW, NREGP), jnp.int32),
            pltpu.VMEM((NREGP + L,), jnp.int32),
            pltpu.VMEM((NROW, 128), jnp.int32),
            pltpu.VMEM((NROW, 128), jnp.int32),
            pltpu.VMEM((256,), jnp.int32),
            pltpu.SemaphoreType.DMA,
        ],
        **_SC_PARAMS,
    )
    def k(row_hbm, col_hbm, et_hbm, regp_hbm, spk_hbm, bnd_hbm,
          rowv, colv, etv, cntv, mystart, posb, pkb, bndv, sem):
        wid = lax.axis_index("s") * NC + lax.axis_index("c")
        base = wid * EPW
        pltpu.sync_copy(row_hbm.at[pl.ds(base, EPW)], rowv)
        pltpu.sync_copy(col_hbm.at[pl.ds(base, EPW)], colv)
        pltpu.sync_copy(et_hbm.at[pl.ds(base, EPW)], etv)
        pltpu.sync_copy(regp_hbm, cntv)

        pos = lax.iota(jnp.int32, L)
        pos_next = jnp.minimum(pos + 1, L - 1)
        pos_prev = jnp.maximum(pos - 1, 0)
        last_lane = jnp.full((L,), L - 1, jnp.int32)

        carry = jnp.zeros((L,), jnp.int32)
        for j in range(NREGP // L):
            sl = pl.ds(j * L, L)
            tot = cntv[0, sl]
            for w in range(1, NW):
                tot = tot + cntv[w, sl]

            def wpre(w, acc, sl=sl):
                return acc + cntv[w, sl]

            mypre = lax.fori_loop(0, wid, wpre, jnp.zeros((L,), jnp.int32))
            incl = plsc.cumsum(tot)
            exc = incl - tot + carry
            carry = carry + _take16(incl, last_lane)
            mystart[sl] = exc + mypre
            bndv[sl] = exc

        @pl.when(wid == 0)
        def _(carry=carry):
            for j in range(NREGP // L, 256 // L):
                bndv[pl.ds(j * L, L)] = carry
            pltpu.sync_copy(bndv, bnd_hbm)

        for b in range(128 // L):
            posb[NROW - 1, pl.ds(b * L, L)] = EXT - 128 + b * L + pos

        def group(gi, ri, b):
            o = gi * L
            cv = colv[pl.ds(o, L)]
            tv = etv[pl.ds(o, L)]
            rv = rowv[pl.ds(o, L)]
            gg = jnp.where(tv < T, tv * NB + lax.shift_right_logical(cv, 8),
                           T * NB)
            yr = jnp.where(tv < T, tv * NPAD + rv, 0)
            pk = yr | lax.shift_left(cv, 17)
            s, p = plsc.sort_key_val(gg, pos)
            rank, is_last = _run_length_split(s, pos, pos_next, pos_prev)
            st = plsc.load_gather(mystart, [s])
            newpos = st + rank
            plsc.store_scatter(mystart, [s], newpos + 1, mask=is_last)
            co = pl.ds(b * L, L)
            posb[ri, co] = newpos
            pkb[ri, co] = _take16(pk, p)

        def abody(ri, _):
            for b in range(8):
                group(ri * 8 + b, ri, b)
            return 0

        lax.fori_loop(0, NGRP // 8, abody, 0)
        for b in range(NGRP % 8):
            group((NGRP // 8) * 8 + b, NGRP // 8, b)

        for kk0 in range(0, NROW, 8):
            for kk in range(kk0, min(kk0 + 8, NROW)):
                pltpu.async_copy(pkb.at[kk], spk_hbm.at[posb.at[kk]], sem)
            for kk in range(kk0, min(kk0 + 8, NROW)):
                pltpu.make_async_copy(
                    pkb.at[kk], spk_hbm.at[posb.at[kk]], sem).wait()

    return k(row, col, et, regp)


def _tc_transform(x_pad, Ws, degp, T, NPAD, H):
    NB = NPAD // BLK

    def body(x_ref, w_ref, deg_ref, y_ref, dinv_ref):
        n = pl.program_id(1)
        dblk = deg_ref[0, :, pl.ds(n * BLK, BLK)]
        deg = jnp.sum(dblk, axis=0).astype(jnp.float32) + 1.0
        dinv = 1.0 / jnp.sqrt(deg)
        xw = lax.dot_general(
            x_ref[...], w_ref[0],
            (((1,), (1,)), ((), ())),
            precision=lax.Precision.HIGHEST,
        )
        y_ref[0] = dinv[:, None] * xw
        dinv_ref[0, :, 0] = dinv

    return pl.pallas_call(
        body,
        grid=(T, NB),
        in_specs=[
            pl.BlockSpec((BLK, H), lambda t, n: (n, 0)),
            pl.BlockSpec((1, H, H), lambda t, n: (t, 0, 0)),
            pl.BlockSpec((1, NW, NPAD), lambda t, n: (t, 0, 0)),
        ],
        out_specs=[
            pl.BlockSpec((1, BLK, H), lambda t, n: (t, n, 0)),
            pl.BlockSpec((1, BLK, 1), lambda t, n: (t, n, 0)),
        ],
        out_shape=[
            jax.ShapeDtypeStruct((T, NPAD, H), jnp.float32),
            jax.ShapeDtypeStruct((T, NPAD, 1), jnp.float32),
        ],
    )(x_pad, Ws, degp)


def _sc_scatter_max(spk, bounds, y3, T, NPAD, H):
    NB = NPAD // BLK
    ROUNDS = (T * NB + NW - 1) // NW
    HV = H // L
    mesh = plsc.VectorSubcoreMesh(core_axis_name="c", subcore_axis_name="s",
                                  num_cores=NC, num_subcores=NS)

    @functools.partial(
        pl.kernel,
        out_type=jax.ShapeDtypeStruct((T * NPAD, H), jnp.float32),
        mesh=mesh,
        scratch_types=[
            pltpu.VMEM((BLK + 1, H), jnp.float32),
            pltpu.VMEM((2, GB + L), jnp.int32),
            pltpu.VMEM((2, GB), jnp.int32),
            pltpu.VMEM((2, GB + L), jnp.int32),
            pltpu.VMEM((2, GB, H), jnp.float32),
            pltpu.VMEM((256,), jnp.int32),
            pltpu.SemaphoreType.DMA,
            pltpu.SemaphoreType.DMA,
        ],
        **_SC_PARAMS,
    )
    def k(spk_hbm, bnd_hbm, y3_hbm, out_hbm,
          accv, pkbuf, sybuf, dvbuf, stag, bndv, isem, gsem):
        wid = lax.axis_index("s") * NC + lax.axis_index("c")
        pltpu.sync_copy(bnd_hbm, bndv)

        def idx_start(a0, k, slot):
            off = pl.multiple_of(a0 + k * GB, 8)
            pltpu.async_copy(spk_hbm.at[pl.ds(off, GB)],
                             pkbuf.at[slot, pl.ds(0, GB)], isem)

        def idx_wait(a0, k, slot):
            off = pl.multiple_of(a0 + k * GB, 8)
            pltpu.make_async_copy(spk_hbm.at[pl.ds(off, GB)],
                                  pkbuf.at[slot, pl.ds(0, GB)], isem).wait()

        def unpack_rows(slot, k, a0, s0, s1, base):
            pos16 = lax.iota(jnp.int32, L)
            for b in range(GB // L):
                sl = pl.ds(b * L, L)
                pk = pkbuf[slot, sl]
                sybuf[slot, sl] = pk & 0x1FFFF
                pg = a0 + k * GB + b * L + pos16
                d = lax.shift_right_logical(pk, 17) - base
                dvbuf[slot, sl] = jnp.where((pg >= s0) & (pg < s1), d, BLK)
            dvbuf[slot, pl.ds(GB, L)] = jnp.full((L,), BLK, jnp.int32)

        def gat_start(slot):
            pltpu.async_copy(y3_hbm.at[sybuf.at[slot]], stag.at[slot], gsem)

        def gat_wait(slot):
            pltpu.make_async_copy(y3_hbm.at[sybuf.at[slot]],
                                  stag.at[slot], gsem).wait()

        for rnd in range(ROUNDS):
            g = wid + NW * rnd
            t = g // NB
            r = g % NB
            base = r * BLK
            ybase = pl.multiple_of(t * NPAD + base, 8)

            s0 = bndv[pl.ds(g, L)][0]
            s1 = bndv[pl.ds(g + 1, L)][0]
            a0 = s0 & (-8)
            nk = (s1 - a0 + GB - 1) // GB

            pltpu.sync_copy(y3_hbm.at[pl.ds(ybase, BLK)],
                            accv.at[pl.ds(0, BLK)])

            @pl.when(nk > 0)
            def _(s0=s0, s1=s1, a0=a0, nk=nk, base=base):
                idx_start(a0, 0, 0)
                idx_wait(a0, 0, 0)
                unpack_rows(0, 0, a0, s0, s1, base)
                gat_start(0)

                @pl.when(nk > 1)
                def _():
                    idx_start(a0, 1, 1)

                def kbody(k, _):
                    par = k & 1
                    opar = 1 - par

                    @pl.when(k + 1 < nk)
                    def _():
                        idx_wait(a0, k + 1, opar)
                        unpack_rows(opar, k + 1, a0, s0, s1, base)
                        gat_start(opar)

                    gat_wait(par)

                    def mbody(j, dcur):
                        dnext = dvbuf[par, pl.ds(j + 1, L)][0]
                        for h in range(HV):
                            sl = pl.ds(h * L, L)
                            accv[dcur, sl] = jnp.maximum(accv[dcur, sl],
                                                         stag[par, j, sl])
                        return dnext

                    lax.fori_loop(0, GB, mbody,
                                  dvbuf[par, pl.ds(0, L)][0])

                    @pl.when(k + 2 < nk)
                    def _():
                        idx_start(a0, k + 2, par)

                    return 0

                lax.fori_loop(0, nk, kbody, 0)

            pltpu.sync_copy(accv.at[pl.ds(0, BLK)], out_hbm.at[pl.ds(ybase, BLK)])

    return k(spk, bounds, y3)


def _tc_ffn(x_pad, acc, dinv, bsum, gamma1, beta1, gamma2, beta2,
            W1, b1, W2, b2, T, NPAD, H, D):
    FB = 512
    NB = NPAD // FB

    def body(x_ref, acc_ref, dinv_ref, bsum_ref, g1_ref, be1_ref,
             g2_ref, be2_ref, w1_ref, b1_ref, w2_ref, b2_ref, out_ref):
        x2 = dinv_ref[0][:, None] * acc_ref[0]
        for t in range(1, T):
            x2 = x2 + dinv_ref[t][:, None] * acc_ref[t]
        h = x_ref[...] + x2 + bsum_ref[0][None, :]
        scale1 = g1_ref[0] * (1.0 / jnp.sqrt(1.0 + 1e-5))
        h = h * scale1[None, :] + be1_ref[0][None, :]
        m1 = lax.dot_general(h, w1_ref[...], (((1,), (1,)), ((), ())),
                             precision=lax.Precision.HIGHEST)
        m1 = jnp.maximum(m1 + b1_ref[0][None, :], 0.0)
        o = lax.dot_general(m1, w2_ref[...], (((1,), (1,)), ((), ())),
                            precision=lax.Precision.HIGHEST)
        o = o + b2_ref[0][None, :]
        scale2 = g2_ref[0] * (1.0 / jnp.sqrt(1.0 + 1e-5))
        out_ref[...] = o * scale2[None, :] + be2_ref[0][None, :]

    return pl.pallas_call(
        body,
        grid=(NB,),
        in_specs=[
            pl.BlockSpec((FB, H), lambda n: (n, 0)),
            pl.BlockSpec((T, FB, H), lambda n: (0, n, 0)),
            pl.BlockSpec((T, FB), lambda n: (0, n)),
            pl.BlockSpec((1, H), lambda n: (0, 0)),
            pl.BlockSpec((1, H), lambda n: (0, 0)),
            pl.BlockSpec((1, H), lambda n: (0, 0)),
            pl.BlockSpec((1, H), lambda n: (0, 0)),
            pl.BlockSpec((1, H), lambda n: (0, 0)),
            pl.BlockSpec((D, H), lambda n: (0, 0)),
            pl.BlockSpec((1, D), lambda n: (0, 0)),
            pl.BlockSpec((H, D), lambda n: (0, 0)),
            pl.BlockSpec((1, H), lambda n: (0, 0)),
        ],
        out_specs=pl.BlockSpec((FB, H), lambda n: (n, 0)),
        out_shape=jax.ShapeDtypeStruct((NPAD, H), jnp.float32),
    )(x_pad, acc, dinv, bsum, gamma1, beta1, gamma2, beta2, W1, b1, W2, b2)


def kernel(x, edge_index, edge_type, Ws, bs, gamma1, beta1, gamma2, beta2,
           W1, b1, W2, b2):
    N, H = x.shape
    T = Ws.shape[0]
    D = W1.shape[0]
    E = edge_type.shape[0]
    NPAD = ((N + BLK - 1) // BLK) * BLK
    NB = NPAD // BLK
    EPAD = ((E + CH - 1) // CH) * CH
    EXT = EPAD + 128

    row = edge_index[0]
    col = edge_index[1]
    if EPAD != E:
        pad = EPAD - E
        row = jnp.concatenate([row, jnp.zeros((pad,), jnp.int32)])
        col = jnp.concatenate([col, jnp.zeros((pad,), jnp.int32)])
        edge_type = jnp.concatenate(
            [edge_type, jnp.full((pad,), T, jnp.int32)])

    x_pad = jnp.pad(x, ((0, NPAD - N), (0, 0)))

    degp, regp = _sc_count(col, edge_type, T, N, NB)
    spk, bounds = _sc_bucket(row, col, edge_type, regp, T, NB, NPAD, EXT)

    degp = degp[:, :T * N].reshape(NW, T, N).transpose(1, 0, 2)
    degp = jnp.pad(degp, ((0, 0), (0, 0), (0, NPAD - N)))

    y, dinv = _tc_transform(x_pad, Ws, degp, T, NPAD, H)
    dinv = dinv[:, :, 0]
    y3 = y.reshape(T * NPAD, H)

    acc = _sc_scatter_max(spk, bounds, y3, T, NPAD, H)
    acc = acc.reshape(T, NPAD, H)

    bsum = jnp.sum(bs, axis=0, keepdims=True)
    out = _tc_ffn(x_pad, acc, dinv, bsum,
                  gamma1[None, :], beta1[None, :],
                  gamma2[None, :], beta2[None, :],
                  W1, b1[None, :], W2, b2[None, :], T, NPAD, H, D)
    return out[:N]


# speedup vs baseline: 1.4353x; 1.4353x over previous
import functools

import jax
import jax.numpy as jnp
from jax import lax
from jax.experimental import pallas as pl
from jax.experimental.pallas import tpu as pltpu
from jax.experimental.pallas import tpu_sc as plsc

NC = 2
NS = 16
NW = NC * NS
L = 16

BLK = 256
CH = 2048
GB = 112
NREGP = 176

_SC_PARAMS = dict(
    compiler_params=pltpu.CompilerParams(needs_layout_passes=False))


def _take16(v, idx):
    return lax.gather(
        v, idx[:, None],
        lax.GatherDimensionNumbers(offset_dims=(), collapsed_slice_dims=(0,),
                                   start_index_map=(0,)),
        (1,), mode=lax.GatherScatterMode.PROMISE_IN_BOUNDS)


def _run_length_split(s, pos, pos_next, pos_prev):
    is_last = (s != _take16(s, pos_next)) | (pos == L - 1)
    is_first = (s != _take16(s, pos_prev)) | (pos == 0)
    fpos = plsc.cummax(jnp.where(is_first, pos, -1))
    return pos - fpos, is_last


def _sc_count(col, et, T, N, NB):
    E = col.shape[0]
    EPW = E // NW
    DSZ = T * N + L
    mesh = plsc.VectorSubcoreMesh(core_axis_name="c", subcore_axis_name="s",
                                  num_cores=NC, num_subcores=NS)

    @functools.partial(
        pl.kernel,
        out_type=(jax.ShapeDtypeStruct((NW, DSZ), jnp.int32),
                  jax.ShapeDtypeStruct((NW, NREGP), jnp.int32)),
        mesh=mesh,
        scratch_types=[
            pltpu.VMEM((EPW,), jnp.int32),
            pltpu.VMEM((EPW,), jnp.int32),
            pltpu.VMEM((DSZ,), jnp.int32),
            pltpu.VMEM((NREGP,), jnp.int32),
        ],
        **_SC_PARAMS,
    )
    def k(col_hbm, et_hbm, deg_hbm, reg_hbm, colv, etv, degv, regv):
        wid = lax.axis_index("s") * NC + lax.axis_index("c")
        base = wid * EPW
        pltpu.sync_copy(col_hbm.at[pl.ds(base, EPW)], colv)
        pltpu.sync_copy(et_hbm.at[pl.ds(base, EPW)], etv)

        zero = jnp.zeros((L,), jnp.int32)

        def zd(i, _):
            degv[pl.ds(i * L, L)] = zero
            return 0

        lax.fori_loop(0, DSZ // L, zd, 0)
        for i in range(NREGP // L):
            regv[pl.ds(i * L, L)] = zero

        pos = lax.iota(jnp.int32, L)
        pos_next = jnp.minimum(pos + 1, L - 1)
        pos_prev = jnp.maximum(pos - 1, 0)

        def hist(tab, keys):
            s, _ = plsc.sort_key_val(keys, keys)
            rank, is_last = _run_length_split(s, pos, pos_next, pos_prev)
            old = plsc.load_gather(tab, [s])
            plsc.store_scatter(tab, [s], old + rank + 1, mask=is_last)

        def sbody(i, _):
            cv = colv[pl.ds(i * L, L)]
            tv = etv[pl.ds(i * L, L)]
            hist(degv, tv * N + cv)
            gg = jnp.where(tv < T, tv * NB + lax.shift_right_logical(cv, 8),
                           T * NB)
            hist(regv, gg)
            return 0

        lax.fori_loop(0, EPW // L, sbody, 0)
        pltpu.sync_copy(degv, deg_hbm.at[wid])
        pltpu.sync_copy(regv, reg_hbm.at[wid])

    return k(col, et)


def _sc_bucket(row, col, et, regp, T, NB, NPAD, EXT):
    E = row.shape[0]
    EPW = E // NW
    NGRP = EPW // L
    NROW = (NGRP * L + 127) // 128
    mesh = plsc.VectorSubcoreMesh(core_axis_name="c", subcore_axis_name="s",
                                  num_cores=NC, num_subcores=NS)

    STG = ((EXT + (NS * 1024) - 1) // (NS * 1024)) * (NS * 1024)
    ZPW = STG // NS

    @functools.partial(
        pl.kernel,
        out_type=(jax.ShapeDtypeStruct((NC, EXT), jnp.int32),
                  jax.ShapeDtypeStruct((256,), jnp.int32)),
        mesh=mesh,
        scratch_types=[
            pltpu.VMEM((EPW,), jnp.int32),
            pltpu.VMEM((EPW,), jnp.int32),
            pltpu.VMEM((EPW,), jnp.int32),
            pltpu.VMEM((NW, NREGP), jnp.int32),
            pltpu.VMEM((NREGP + L,), jnp.int32),
            pltpu.VMEM((NROW, 128), jnp.int32),
            pltpu.VMEM((NROW, 128), jnp.int32),
            pltpu.VMEM((256,), jnp.int32),
            pltpu.VMEM((1024,), jnp.int32),
            pltpu.VMEM_SHARED((STG,), jnp.int32),
            pltpu.SemaphoreType.DMA,
        ],
        **_SC_PARAMS,
    )
    def k(row_hbm, col_hbm, et_hbm, regp_hbm, spk_hbm, bnd_hbm,
          rowv, colv, etv, cntv, mystart, posb, pkb, bndv, zb, stage, sem):
        cid = lax.axis_index("c")
        sid = lax.axis_index("s")
        wid = lax.axis_index("s") * NC + lax.axis_index("c")

        neg1 = jnp.full((L,), -1, jnp.int32)
        for i in range(1024 // L):
            zb[pl.ds(i * L, L)] = neg1
        for z in range(ZPW // 1024):
            pltpu.async_copy(
                zb, stage.at[pl.ds(sid * ZPW + z * 1024, 1024)], sem)
        for z in range(ZPW // 1024):
            pltpu.make_async_copy(
                zb, stage.at[pl.ds(sid * ZPW + z * 1024, 1024)], sem).wait()
        base = wid * EPW
        pltpu.sync_copy(row_hbm.at[pl.ds(base, EPW)], rowv)
        pltpu.sync_copy(col_hbm.at[pl.ds(base, EPW)], colv)
        pltpu.sync_copy(et_hbm.at[pl.ds(base, EPW)], etv)
        pltpu.sync_copy(regp_hbm, cntv)

        pos = lax.iota(jnp.int32, L)
        pos_next = jnp.minimum(pos + 1, L - 1)
        pos_prev = jnp.maximum(pos - 1, 0)
        last_lane = jnp.full((L,), L - 1, jnp.int32)

        carry = jnp.zeros((L,), jnp.int32)
        for j in range(NREGP // L):
            sl = pl.ds(j * L, L)
            tot = cntv[0, sl]
            for w in range(1, NW):
                tot = tot + cntv[w, sl]

            def wpre(w, acc, sl=sl):
                return acc + cntv[w, sl]

            mypre = lax.fori_loop(0, wid, wpre, jnp.zeros((L,), jnp.int32))
            incl = plsc.cumsum(tot)
            exc = incl - tot + carry
            carry = carry + _take16(incl, last_lane)
            mystart[sl] = exc + mypre
            bndv[sl] = exc

        @pl.when(wid == 0)
        def _(carry=carry):
            for j in range(NREGP // L, 256 // L):
                bndv[pl.ds(j * L, L)] = carry
            pltpu.sync_copy(bndv, bnd_hbm)

        for b in range(128 // L):
            posb[NROW - 1, pl.ds(b * L, L)] = EXT - 128 + b * L + pos

        def group(gi, ri, b):
            o = gi * L
            cv = colv[pl.ds(o, L)]
            tv = etv[pl.ds(o, L)]
            rv = rowv[pl.ds(o, L)]
            gg = jnp.where(tv < T, tv * NB + lax.shift_right_logical(cv, 8),
                           T * NB)
            yr = jnp.where(tv < T, tv * NPAD + rv, 0)
            pk = yr | lax.shift_left(cv, 17)
            s, p = plsc.sort_key_val(gg, pos)
            rank, is_last = _run_length_split(s, pos, pos_next, pos_prev)
            st = plsc.load_gather(mystart, [s])
            newpos = st + rank
            plsc.store_scatter(mystart, [s], newpos + 1, mask=is_last)
            co = pl.ds(b * L, L)
            posb[ri, co] = newpos
            pkb[ri, co] = _take16(pk, p)

        def abody(ri, _):
            for b in range(8):
                group(ri * 8 + b, ri, b)
            return 0

        lax.fori_loop(0, NGRP // 8, abody, 0)
        for b in range(NGRP % 8):
            group((NGRP // 8) * 8 + b, NGRP // 8, b)

        plsc.subcore_barrier()

        for kk0 in range(0, NROW, 8):
            for kk in range(kk0, min(kk0 + 8, NROW)):
                pltpu.async_copy(pkb.at[kk], stage.at[posb.at[kk]], sem)
            for kk in range(kk0, min(kk0 + 8, NROW)):
                pltpu.make_async_copy(
                    pkb.at[kk], stage.at[posb.at[kk]], sem).wait()

        plsc.subcore_barrier()

        @pl.when(sid == 0)
        def _():
            pltpu.sync_copy(stage.at[pl.ds(0, EXT)], spk_hbm.at[cid])

    return k(row, col, et, regp)


def _tc_transform(x_pad, Ws, degp, T, NPAD, H):
    NB = NPAD // BLK

    def body(x_ref, w_ref, deg_ref, y_ref, dinv_ref):
        n = pl.program_id(1)
        dblk = deg_ref[0, :, pl.ds(n * BLK, BLK)]
        deg = jnp.sum(dblk, axis=0).astype(jnp.float32) + 1.0
        dinv = 1.0 / jnp.sqrt(deg)
        xw = lax.dot_general(
            x_ref[...], w_ref[0],
            (((1,), (1,)), ((), ())),
            precision=lax.Precision.HIGHEST,
        )
        y_ref[0] = dinv[:, None] * xw
        dinv_ref[0, :, 0] = dinv

    return pl.pallas_call(
        body,
        grid=(T, NB),
        in_specs=[
            pl.BlockSpec((BLK, H), lambda t, n: (n, 0)),
            pl.BlockSpec((1, H, H), lambda t, n: (t, 0, 0)),
            pl.BlockSpec((1, NW, NPAD), lambda t, n: (t, 0, 0)),
        ],
        out_specs=[
            pl.BlockSpec((1, BLK, H), lambda t, n: (t, n, 0)),
            pl.BlockSpec((1, BLK, 1), lambda t, n: (t, n, 0)),
        ],
        out_shape=[
            jax.ShapeDtypeStruct((T, NPAD, H), jnp.float32),
            jax.ShapeDtypeStruct((T, NPAD, 1), jnp.float32),
        ],
    )(x_pad, Ws, degp)


def _sc_scatter_max(spk, bounds, y3, T, NPAD, H):
    NB = NPAD // BLK
    ROUNDS = (T * NB + NW - 1) // NW
    HV = H // L
    mesh = plsc.VectorSubcoreMesh(core_axis_name="c", subcore_axis_name="s",
                                  num_cores=NC, num_subcores=NS)

    @functools.partial(
        pl.kernel,
        out_type=jax.ShapeDtypeStruct((T * NPAD, H), jnp.float32),
        mesh=mesh,
        scratch_types=[
            pltpu.VMEM((BLK + 1, H), jnp.float32),
            pltpu.VMEM((2, GB + L), jnp.int32),
            pltpu.VMEM((2, GB), jnp.int32),
            pltpu.VMEM((2, GB + L), jnp.int32),
            pltpu.VMEM((2, GB, H), jnp.float32),
            pltpu.VMEM((256,), jnp.int32),
            pltpu.SemaphoreType.DMA,
            pltpu.SemaphoreType.DMA,
        ],
        **_SC_PARAMS,
    )
    def k(spk_hbm, bnd_hbm, y3_hbm, out_hbm,
          accv, pkbuf, sybuf, dvbuf, stag, bndv, isem, gsem):
        wid = lax.axis_index("s") * NC + lax.axis_index("c")
        pltpu.sync_copy(bnd_hbm, bndv)

        def idx_start(a0, k, slot):
            off = pl.multiple_of(a0 + k * GB, 8)
            pltpu.async_copy(spk_hbm.at[pl.ds(off, GB)],
                             pkbuf.at[slot, pl.ds(0, GB)], isem)

        def idx_wait(a0, k, slot):
            off = pl.multiple_of(a0 + k * GB, 8)
            pltpu.make_async_copy(spk_hbm.at[pl.ds(off, GB)],
                                  pkbuf.at[slot, pl.ds(0, GB)], isem).wait()

        def unpack_rows(slot, k, a0, s0, s1, base):
            pos16 = lax.iota(jnp.int32, L)
            for b in range(GB // L):
                sl = pl.ds(b * L, L)
                pk = pkbuf[slot, sl]
                sybuf[slot, sl] = pk & 0x1FFFF
                pg = a0 + k * GB + b * L + pos16
                d = lax.shift_right_logical(pk, 17) - base
                dvbuf[slot, sl] = jnp.where((pg >= s0) & (pg < s1), d, BLK)
            dvbuf[slot, pl.ds(GB, L)] = jnp.full((L,), BLK, jnp.int32)

        def gat_start(slot):
            pltpu.async_copy(y3_hbm.at[sybuf.at[slot]], stag.at[slot], gsem)

        def gat_wait(slot):
            pltpu.make_async_copy(y3_hbm.at[sybuf.at[slot]],
                                  stag.at[slot], gsem).wait()

        for rnd in range(ROUNDS):
            g = wid + NW * rnd
            t = g // NB
            r = g % NB
            base = r * BLK
            ybase = pl.multiple_of(t * NPAD + base, 8)

            s0 = bndv[pl.ds(g, L)][0]
            s1 = bndv[pl.ds(g + 1, L)][0]
            a0 = s0 & (-8)
            nk = (s1 - a0 + GB - 1) // GB

            pltpu.sync_copy(y3_hbm.at[pl.ds(ybase, BLK)],
                            accv.at[pl.ds(0, BLK)])

            @pl.when(nk > 0)
            def _(s0=s0, s1=s1, a0=a0, nk=nk, base=base):
                idx_start(a0, 0, 0)
                idx_wait(a0, 0, 0)
                unpack_rows(0, 0, a0, s0, s1, base)
                gat_start(0)

                @pl.when(nk > 1)
                def _():
                    idx_start(a0, 1, 1)

                def kbody(k, _):
                    par = k & 1
                    opar = 1 - par

                    @pl.when(k + 1 < nk)
                    def _():
                        idx_wait(a0, k + 1, opar)
                        unpack_rows(opar, k + 1, a0, s0, s1, base)
                        gat_start(opar)

                    gat_wait(par)

                    def mbody(j, dcur):
                        dnext = dvbuf[par, pl.ds(j + 1, L)][0]
                        for h in range(HV):
                            sl = pl.ds(h * L, L)
                            accv[dcur, sl] = jnp.maximum(accv[dcur, sl],
                                                         stag[par, j, sl])
                        return dnext

                    lax.fori_loop(0, GB, mbody,
                                  dvbuf[par, pl.ds(0, L)][0])

                    @pl.when(k + 2 < nk)
                    def _():
                        idx_start(a0, k + 2, par)

                    return 0

                lax.fori_loop(0, nk, kbody, 0)

            pltpu.sync_copy(accv.at[pl.ds(0, BLK)], out_hbm.at[pl.ds(ybase, BLK)])

    return k(spk, bounds, y3)


def _tc_ffn(x_pad, acc, dinv, bsum, gamma1, beta1, gamma2, beta2,
            W1, b1, W2, b2, T, NPAD, H, D):
    FB = 512
    NB = NPAD // FB

    def body(x_ref, acc_ref, dinv_ref, bsum_ref, g1_ref, be1_ref,
             g2_ref, be2_ref, w1_ref, b1_ref, w2_ref, b2_ref, out_ref):
        x2 = dinv_ref[0][:, None] * acc_ref[0]
        for t in range(1, T):
            x2 = x2 + dinv_ref[t][:, None] * acc_ref[t]
        h = x_ref[...] + x2 + bsum_ref[0][None, :]
        scale1 = g1_ref[0] * (1.0 / jnp.sqrt(1.0 + 1e-5))
        h = h * scale1[None, :] + be1_ref[0][None, :]
        m1 = lax.dot_general(h, w1_ref[...], (((1,), (1,)), ((), ())),
                             precision=lax.Precision.HIGHEST)
        m1 = jnp.maximum(m1 + b1_ref[0][None, :], 0.0)
        o = lax.dot_general(m1, w2_ref[...], (((1,), (1,)), ((), ())),
                            precision=lax.Precision.HIGHEST)
        o = o + b2_ref[0][None, :]
        scale2 = g2_ref[0] * (1.0 / jnp.sqrt(1.0 + 1e-5))
        out_ref[...] = o * scale2[None, :] + be2_ref[0][None, :]

    return pl.pallas_call(
        body,
        grid=(NB,),
        in_specs=[
            pl.BlockSpec((FB, H), lambda n: (n, 0)),
            pl.BlockSpec((T, FB, H), lambda n: (0, n, 0)),
            pl.BlockSpec((T, FB), lambda n: (0, n)),
            pl.BlockSpec((1, H), lambda n: (0, 0)),
            pl.BlockSpec((1, H), lambda n: (0, 0)),
            pl.BlockSpec((1, H), lambda n: (0, 0)),
            pl.BlockSpec((1, H), lambda n: (0, 0)),
            pl.BlockSpec((1, H), lambda n: (0, 0)),
            pl.BlockSpec((D, H), lambda n: (0, 0)),
            pl.BlockSpec((1, D), lambda n: (0, 0)),
            pl.BlockSpec((H, D), lambda n: (0, 0)),
            pl.BlockSpec((1, H), lambda n: (0, 0)),
        ],
        out_specs=pl.BlockSpec((FB, H), lambda n: (n, 0)),
        out_shape=jax.ShapeDtypeStruct((NPAD, H), jnp.float32),
    )(x_pad, acc, dinv, bsum, gamma1, beta1, gamma2, beta2, W1, b1, W2, b2)


def kernel(x, edge_index, edge_type, Ws, bs, gamma1, beta1, gamma2, beta2,
           W1, b1, W2, b2):
    N, H = x.shape
    T = Ws.shape[0]
    D = W1.shape[0]
    E = edge_type.shape[0]
    NPAD = ((N + BLK - 1) // BLK) * BLK
    NB = NPAD // BLK
    EPAD = ((E + CH - 1) // CH) * CH
    EXT = EPAD + 128

    row = edge_index[0]
    col = edge_index[1]
    if EPAD != E:
        pad = EPAD - E
        row = jnp.concatenate([row, jnp.zeros((pad,), jnp.int32)])
        col = jnp.concatenate([col, jnp.zeros((pad,), jnp.int32)])
        edge_type = jnp.concatenate(
            [edge_type, jnp.full((pad,), T, jnp.int32)])

    x_pad = jnp.pad(x, ((0, NPAD - N), (0, 0)))

    degp, regp = _sc_count(col, edge_type, T, N, NB)
    spk2, bounds = _sc_bucket(row, col, edge_type, regp, T, NB, NPAD, EXT)
    spk = jnp.where(spk2[0] >= 0, spk2[0], spk2[1])

    degp = degp[:, :T * N].reshape(NW, T, N).transpose(1, 0, 2)
    degp = jnp.pad(degp, ((0, 0), (0, 0), (0, NPAD - N)))

    y, dinv = _tc_transform(x_pad, Ws, degp, T, NPAD, H)
    dinv = dinv[:, :, 0]
    y3 = y.reshape(T * NPAD, H)

    acc = _sc_scatter_max(spk, bounds, y3, T, NPAD, H)
    acc = acc.reshape(T, NPAD, H)

    bsum = jnp.sum(bs, axis=0, keepdims=True)
    out = _tc_ffn(x_pad, acc, dinv, bsum,
                  gamma1[None, :], beta1[None, :],
                  gamma2[None, :], beta2[None, :],
                  W1, b1[None, :], W2, b2[None, :], T, NPAD, H, D)
    return out[:N]


# speedup vs baseline: 1.4531x; 1.0124x over previous
import functools

import jax
import jax.numpy as jnp
from jax import lax
from jax.experimental import pallas as pl
from jax.experimental.pallas import tpu as pltpu
from jax.experimental.pallas import tpu_sc as plsc

NC = 2
NS = 16
NW = NC * NS
L = 16

BLK = 256
CH = 2048
GB = 112
NREGP = 176

_SC_PARAMS = dict(
    compiler_params=pltpu.CompilerParams(needs_layout_passes=False))


def _take16(v, idx):
    return lax.gather(
        v, idx[:, None],
        lax.GatherDimensionNumbers(offset_dims=(), collapsed_slice_dims=(0,),
                                   start_index_map=(0,)),
        (1,), mode=lax.GatherScatterMode.PROMISE_IN_BOUNDS)


def _run_length_split(s, pos, pos_next, pos_prev):
    is_last = (s != _take16(s, pos_next)) | (pos == L - 1)
    is_first = (s != _take16(s, pos_prev)) | (pos == 0)
    fpos = plsc.cummax(jnp.where(is_first, pos, -1))
    return pos - fpos, is_last


def _sc_count(col, et, T, N, NB):
    E = col.shape[0]
    EPW = E // NW
    DSZ = T * N + L
    mesh = plsc.VectorSubcoreMesh(core_axis_name="c", subcore_axis_name="s",
                                  num_cores=NC, num_subcores=NS)

    @functools.partial(
        pl.kernel,
        out_type=(jax.ShapeDtypeStruct((NW, DSZ), jnp.int32),
                  jax.ShapeDtypeStruct((NW, NREGP), jnp.int32)),
        mesh=mesh,
        scratch_types=[
            pltpu.VMEM((EPW,), jnp.int32),
            pltpu.VMEM((EPW,), jnp.int32),
            pltpu.VMEM((DSZ,), jnp.int32),
            pltpu.VMEM((NREGP,), jnp.int32),
        ],
        **_SC_PARAMS,
    )
    def k(col_hbm, et_hbm, deg_hbm, reg_hbm, colv, etv, degv, regv):
        wid = lax.axis_index("s") * NC + lax.axis_index("c")
        base = wid * EPW
        pltpu.sync_copy(col_hbm.at[pl.ds(base, EPW)], colv)
        pltpu.sync_copy(et_hbm.at[pl.ds(base, EPW)], etv)

        zero = jnp.zeros((L,), jnp.int32)

        def zd(i, _):
            degv[pl.ds(i * L, L)] = zero
            return 0

        lax.fori_loop(0, DSZ // L, zd, 0)
        for i in range(NREGP // L):
            regv[pl.ds(i * L, L)] = zero

        pos = lax.iota(jnp.int32, L)
        pos_next = jnp.minimum(pos + 1, L - 1)
        pos_prev = jnp.maximum(pos - 1, 0)

        def hist(tab, keys):
            s, _ = plsc.sort_key_val(keys, keys)
            rank, is_last = _run_length_split(s, pos, pos_next, pos_prev)
            old = plsc.load_gather(tab, [s])
            plsc.store_scatter(tab, [s], old + rank + 1, mask=is_last)

        def sbody(i, _):
            cv = colv[pl.ds(i * L, L)]
            tv = etv[pl.ds(i * L, L)]
            hist(degv, tv * N + cv)
            gg = jnp.where(tv < T, tv * NB + lax.shift_right_logical(cv, 8),
                           T * NB)
            hist(regv, gg)
            return 0

        lax.fori_loop(0, EPW // L, sbody, 0)
        pltpu.sync_copy(degv, deg_hbm.at[wid])
        pltpu.sync_copy(regv, reg_hbm.at[wid])

    return k(col, et)


def _sc_bucket(row, col, et, regp, T, NB, NPAD, EXT):
    E = row.shape[0]
    EPW = E // NW
    NGRP = EPW // L
    NROW = (NGRP * L + 127) // 128
    mesh = plsc.VectorSubcoreMesh(core_axis_name="c", subcore_axis_name="s",
                                  num_cores=NC, num_subcores=NS)

    STG = ((EXT + (NS * 1024) - 1) // (NS * 1024)) * (NS * 1024)
    ZPW = STG // NS

    @functools.partial(
        pl.kernel,
        out_type=(jax.ShapeDtypeStruct((NC, EXT), jnp.int32),
                  jax.ShapeDtypeStruct((256,), jnp.int32)),
        mesh=mesh,
        scratch_types=[
            pltpu.VMEM((EPW,), jnp.int32),
            pltpu.VMEM((EPW,), jnp.int32),
            pltpu.VMEM((EPW,), jnp.int32),
            pltpu.VMEM((NW, NREGP), jnp.int32),
            pltpu.VMEM((NREGP + L,), jnp.int32),
            pltpu.VMEM((NROW, 128), jnp.int32),
            pltpu.VMEM((NROW, 128), jnp.int32),
            pltpu.VMEM((256,), jnp.int32),
            pltpu.VMEM((1024,), jnp.int32),
            pltpu.VMEM_SHARED((STG,), jnp.int32),
            pltpu.SemaphoreType.DMA,
        ],
        **_SC_PARAMS,
    )
    def k(row_hbm, col_hbm, et_hbm, regp_hbm, spk_hbm, bnd_hbm,
          rowv, colv, etv, cntv, mystart, posb, pkb, bndv, zb, stage, sem):
        cid = lax.axis_index("c")
        sid = lax.axis_index("s")
        wid = lax.axis_index("s") * NC + lax.axis_index("c")

        neg1 = jnp.full((L,), -1, jnp.int32)
        for i in range(1024 // L):
            zb[pl.ds(i * L, L)] = neg1
        for z in range(ZPW // 1024):
            pltpu.async_copy(
                zb, stage.at[pl.ds(sid * ZPW + z * 1024, 1024)], sem)
        for z in range(ZPW // 1024):
            pltpu.make_async_copy(
                zb, stage.at[pl.ds(sid * ZPW + z * 1024, 1024)], sem).wait()
        base = wid * EPW
        pltpu.sync_copy(row_hbm.at[pl.ds(base, EPW)], rowv)
        pltpu.sync_copy(col_hbm.at[pl.ds(base, EPW)], colv)
        pltpu.sync_copy(et_hbm.at[pl.ds(base, EPW)], etv)
        pltpu.sync_copy(regp_hbm, cntv)

        pos = lax.iota(jnp.int32, L)
        pos_next = jnp.minimum(pos + 1, L - 1)
        pos_prev = jnp.maximum(pos - 1, 0)
        last_lane = jnp.full((L,), L - 1, jnp.int32)

        carry = jnp.zeros((L,), jnp.int32)
        for j in range(NREGP // L):
            sl = pl.ds(j * L, L)
            tot = cntv[0, sl]
            for w in range(1, NW):
                tot = tot + cntv[w, sl]

            def wpre(w, acc, sl=sl):
                return acc + cntv[w, sl]

            mypre = lax.fori_loop(0, wid, wpre, jnp.zeros((L,), jnp.int32))
            incl = plsc.cumsum(tot)
            exc = incl - tot + carry
            carry = carry + _take16(incl, last_lane)
            mystart[sl] = exc + mypre
            bndv[sl] = exc

        @pl.when(wid == 0)
        def _(carry=carry):
            for j in range(NREGP // L, 256 // L):
                bndv[pl.ds(j * L, L)] = carry
            pltpu.sync_copy(bndv, bnd_hbm)

        for b in range(128 // L):
            posb[NROW - 1, pl.ds(b * L, L)] = EXT - 128 + b * L + pos

        def group(gi, ri, b):
            o = gi * L
            cv = colv[pl.ds(o, L)]
            tv = etv[pl.ds(o, L)]
            rv = rowv[pl.ds(o, L)]
            gg = jnp.where(tv < T, tv * NB + lax.shift_right_logical(cv, 8),
                           T * NB)
            yr = jnp.where(tv < T, tv * NPAD + rv, 0)
            pk = yr | lax.shift_left(cv, 17)
            s, p = plsc.sort_key_val(gg, pos)
            rank, is_last = _run_length_split(s, pos, pos_next, pos_prev)
            st = plsc.load_gather(mystart, [s])
            newpos = st + rank
            plsc.store_scatter(mystart, [s], newpos + 1, mask=is_last)
            co = pl.ds(b * L, L)
            posb[ri, co] = newpos
            pkb[ri, co] = _take16(pk, p)

        def abody(ri, _):
            for b in range(8):
                group(ri * 8 + b, ri, b)
            return 0

        lax.fori_loop(0, NGRP // 8, abody, 0)
        for b in range(NGRP % 8):
            group((NGRP // 8) * 8 + b, NGRP // 8, b)

        plsc.subcore_barrier()

        for kk0 in range(0, NROW, 8):
            for kk in range(kk0, min(kk0 + 8, NROW)):
                pltpu.async_copy(pkb.at[kk], stage.at[posb.at[kk]], sem)
            for kk in range(kk0, min(kk0 + 8, NROW)):
                pltpu.make_async_copy(
                    pkb.at[kk], stage.at[posb.at[kk]], sem).wait()

        plsc.subcore_barrier()

        @pl.when(sid == 0)
        def _():
            pltpu.sync_copy(stage.at[pl.ds(0, EXT)], spk_hbm.at[cid])

    return k(row, col, et, regp)


def _tc_transform(x_pad, Ws, degp, T, NPAD, H):
    NB = NPAD // BLK

    def body(x_ref, w_ref, deg_ref, y_ref, dinv_ref):
        n = pl.program_id(1)
        dblk = deg_ref[0, :, pl.ds(n * BLK, BLK)]
        deg = jnp.sum(dblk, axis=0).astype(jnp.float32) + 1.0
        dinv = 1.0 / jnp.sqrt(deg)
        xw = lax.dot_general(
            x_ref[...], w_ref[0],
            (((1,), (1,)), ((), ())),
            precision=lax.Precision.HIGHEST,
        )
        y_ref[0] = dinv[:, None] * xw
        dinv_ref[0, :, 0] = dinv

    return pl.pallas_call(
        body,
        grid=(T, NB),
        in_specs=[
            pl.BlockSpec((BLK, H), lambda t, n: (n, 0)),
            pl.BlockSpec((1, H, H), lambda t, n: (t, 0, 0)),
            pl.BlockSpec((1, NW, NPAD), lambda t, n: (t, 0, 0)),
        ],
        out_specs=[
            pl.BlockSpec((1, BLK, H), lambda t, n: (t, n, 0)),
            pl.BlockSpec((1, BLK, 1), lambda t, n: (t, n, 0)),
        ],
        out_shape=[
            jax.ShapeDtypeStruct((T, NPAD, H), jnp.float32),
            jax.ShapeDtypeStruct((T, NPAD, 1), jnp.float32),
        ],
    )(x_pad, Ws, degp)


def _sc_scatter_max(spk, bounds, y3, T, NPAD, H):
    NB = NPAD // BLK
    ROUNDS = (T * NB + NW - 1) // NW
    HV = H // L
    mesh = plsc.VectorSubcoreMesh(core_axis_name="c", subcore_axis_name="s",
                                  num_cores=NC, num_subcores=NS)

    @functools.partial(
        pl.kernel,
        out_type=jax.ShapeDtypeStruct((T * NPAD, H), jnp.float32),
        mesh=mesh,
        scratch_types=[
            pltpu.VMEM((BLK + 1, H), jnp.float32),
            pltpu.VMEM((2, GB + L), jnp.int32),
            pltpu.VMEM((2, GB), jnp.int32),
            pltpu.VMEM((2, GB + L), jnp.int32),
            pltpu.VMEM((2, GB, H), jnp.float32),
            pltpu.VMEM((256,), jnp.int32),
            pltpu.SemaphoreType.DMA,
            pltpu.SemaphoreType.DMA,
        ],
        **_SC_PARAMS,
    )
    def k(spk_hbm, bnd_hbm, y3_hbm, out_hbm,
          accv, pkbuf, sybuf, dvbuf, stag, bndv, isem, gsem):
        wid = lax.axis_index("s") * NC + lax.axis_index("c")
        pltpu.sync_copy(bnd_hbm, bndv)

        def idx_start(a0, k, slot):
            off = pl.multiple_of(a0 + k * GB, 8)
            pltpu.async_copy(spk_hbm.at[pl.ds(off, GB)],
                             pkbuf.at[slot, pl.ds(0, GB)], isem)

        def idx_wait(a0, k, slot):
            off = pl.multiple_of(a0 + k * GB, 8)
            pltpu.make_async_copy(spk_hbm.at[pl.ds(off, GB)],
                                  pkbuf.at[slot, pl.ds(0, GB)], isem).wait()

        def unpack_rows(slot, k, a0, s0, s1, base):
            pos16 = lax.iota(jnp.int32, L)
            for b in range(GB // L):
                sl = pl.ds(b * L, L)
                pk = pkbuf[slot, sl]
                sybuf[slot, sl] = pk & 0x1FFFF
                pg = a0 + k * GB + b * L + pos16
                d = lax.shift_right_logical(pk, 17) - base
                dvbuf[slot, sl] = jnp.where((pg >= s0) & (pg < s1), d, BLK)
            dvbuf[slot, pl.ds(GB, L)] = jnp.full((L,), BLK, jnp.int32)

        def gat_start(slot):
            pltpu.async_copy(y3_hbm.at[sybuf.at[slot]], stag.at[slot], gsem)

        def gat_wait(slot):
            pltpu.make_async_copy(y3_hbm.at[sybuf.at[slot]],
                                  stag.at[slot], gsem).wait()

        for rnd in range(ROUNDS):
            g = wid + NW * rnd
            t = g // NB
            r = g % NB
            base = r * BLK
            ybase = pl.multiple_of(t * NPAD + base, 8)

            s0 = bndv[pl.ds(g, L)][0]
            s1 = bndv[pl.ds(g + 1, L)][0]
            a0 = s0 & (-8)
            nk = (s1 - a0 + GB - 1) // GB

            pltpu.sync_copy(y3_hbm.at[pl.ds(ybase, BLK)],
                            accv.at[pl.ds(0, BLK)])

            @pl.when(nk > 0)
            def _(s0=s0, s1=s1, a0=a0, nk=nk, base=base):
                idx_start(a0, 0, 0)
                idx_wait(a0, 0, 0)
                unpack_rows(0, 0, a0, s0, s1, base)
                gat_start(0)

                @pl.when(nk > 1)
                def _():
                    idx_start(a0, 1, 1)

                def kbody(k, _):
                    par = k & 1
                    opar = 1 - par

                    @pl.when(k + 1 < nk)
                    def _():
                        idx_wait(a0, k + 1, opar)
                        unpack_rows(opar, k + 1, a0, s0, s1, base)
                        gat_start(opar)

                    gat_wait(par)

                    def mbody(j, dcur):
                        dnext = dvbuf[par, pl.ds(j + 1, L)][0]
                        for h in range(HV):
                            sl = pl.ds(h * L, L)
                            accv[dcur, sl] = jnp.maximum(accv[dcur, sl],
                                                         stag[par, j, sl])
                        return dnext

                    lax.fori_loop(0, GB, mbody,
                                  dvbuf[par, pl.ds(0, L)][0])

                    @pl.when(k + 2 < nk)
                    def _():
                        idx_start(a0, k + 2, par)

                    return 0

                lax.fori_loop(0, nk, kbody, 0)

            pltpu.sync_copy(accv.at[pl.ds(0, BLK)], out_hbm.at[pl.ds(ybase, BLK)])

    return k(spk, bounds, y3)


def _tc_ffn(x, acc, dinv, bsum, gamma1, beta1, gamma2, beta2,
            W1, b1, W2, b2, T, N, H, D):
    FB = 400
    NB = N // FB

    def body(x_ref, acc_ref, dinv_ref, bsum_ref, g1_ref, be1_ref,
             g2_ref, be2_ref, w1_ref, b1_ref, w2_ref, b2_ref, out_ref):
        x2 = dinv_ref[0, :, :] * acc_ref[0]
        for t in range(1, T):
            x2 = x2 + dinv_ref[t, :, :] * acc_ref[t]
        h = x_ref[...] + x2 + bsum_ref[0][None, :]
        scale1 = g1_ref[0] * (1.0 / jnp.sqrt(1.0 + 1e-5))
        h = h * scale1[None, :] + be1_ref[0][None, :]
        m1 = lax.dot_general(h, w1_ref[...], (((1,), (1,)), ((), ())),
                             precision=lax.Precision.HIGHEST)
        m1 = jnp.maximum(m1 + b1_ref[0][None, :], 0.0)
        o = lax.dot_general(m1, w2_ref[...], (((1,), (1,)), ((), ())),
                            precision=lax.Precision.HIGHEST)
        o = o + b2_ref[0][None, :]
        scale2 = g2_ref[0] * (1.0 / jnp.sqrt(1.0 + 1e-5))
        out_ref[...] = o * scale2[None, :] + be2_ref[0][None, :]

    return pl.pallas_call(
        body,
        grid=(NB,),
        in_specs=[
            pl.BlockSpec((FB, H), lambda n: (n, 0)),
            pl.BlockSpec((T, FB, H), lambda n: (0, n, 0)),
            pl.BlockSpec((T, FB, 1), lambda n: (0, n, 0)),
            pl.BlockSpec((1, H), lambda n: (0, 0)),
            pl.BlockSpec((1, H), lambda n: (0, 0)),
            pl.BlockSpec((1, H), lambda n: (0, 0)),
            pl.BlockSpec((1, H), lambda n: (0, 0)),
            pl.BlockSpec((1, H), lambda n: (0, 0)),
            pl.BlockSpec((D, H), lambda n: (0, 0)),
            pl.BlockSpec((1, D), lambda n: (0, 0)),
            pl.BlockSpec((H, D), lambda n: (0, 0)),
            pl.BlockSpec((1, H), lambda n: (0, 0)),
        ],
        out_specs=pl.BlockSpec((FB, H), lambda n: (n, 0)),
        out_shape=jax.ShapeDtypeStruct((N, H), jnp.float32),
    )(x, acc, dinv, bsum, gamma1, beta1, gamma2, beta2, W1, b1, W2, b2)


def kernel(x, edge_index, edge_type, Ws, bs, gamma1, beta1, gamma2, beta2,
           W1, b1, W2, b2):
    N, H = x.shape
    T = Ws.shape[0]
    D = W1.shape[0]
    E = edge_type.shape[0]
    NPAD = ((N + BLK - 1) // BLK) * BLK
    NB = NPAD // BLK
    EPAD = ((E + CH - 1) // CH) * CH
    EXT = EPAD + 128

    row = edge_index[0]
    col = edge_index[1]
    if EPAD != E:
        pad = EPAD - E
        row = jnp.concatenate([row, jnp.zeros((pad,), jnp.int32)])
        col = jnp.concatenate([col, jnp.zeros((pad,), jnp.int32)])
        edge_type = jnp.concatenate(
            [edge_type, jnp.full((pad,), T, jnp.int32)])

    x_pad = jnp.pad(x, ((0, NPAD - N), (0, 0)))

    degp, regp = _sc_count(col, edge_type, T, N, NB)
    spk2, bounds = _sc_bucket(row, col, edge_type, regp, T, NB, NPAD, EXT)
    spk = jnp.where(spk2[0] >= 0, spk2[0], spk2[1])

    degp = degp[:, :T * N].reshape(NW, T, N).transpose(1, 0, 2)
    degp = jnp.pad(degp, ((0, 0), (0, 0), (0, NPAD - N)))

    y, dinv = _tc_transform(x_pad, Ws, degp, T, NPAD, H)
    y3 = y.reshape(T * NPAD, H)

    acc = _sc_scatter_max(spk, bounds, y3, T, NPAD, H)
    acc = acc.reshape(T, NPAD, H)

    bsum = jnp.sum(bs, axis=0, keepdims=True)
    out = _tc_ffn(x, acc, dinv, bsum,
                  gamma1[None, :], beta1[None, :],
                  gamma2[None, :], beta2[None, :],
                  W1, b1[None, :], W2, b2[None, :], T, N, H, D)
    return out
